# Initial kernel scaffold; baseline (speedup 1.0000x reference)
import jax
import jax.numpy as jnp
from jax.experimental import pallas as pl


def kernel(batch_h, batch_t, batch_r, task_mode, mode, ent_re, ent_im, ent_emb, rel_re, rel_im, rel_emb, visual, Wp, bp):
    raise NotImplementedError("write your pallas kernel here")


# speedup vs baseline: 1.4102x; 1.4102x over previous
import functools

import jax
import jax.numpy as jnp
from jax import lax
from jax.experimental import pallas as pl
from jax.experimental.pallas import tpu as pltpu

B = 16384
DIM = 128
VIS = 4096
BM = 256
NB = B // BM


def _score_block(xh_ref, xt_ref, hre_ref, him_ref, h_ref, tre_ref, tim_ref,
                 t_ref, rre_ref, rim_ref, r_ref, tm_ref, wp_ref, bp_ref,
                 out_ref):
    xh = xh_ref[...]
    xt = xt_ref[...]
    wp = wp_ref[...]
    bp = bp_ref[...]
    ha = lax.dot_general(xh, wp, (((1,), (1,)), ((), ())),
                         preferred_element_type=jnp.float32) + bp
    ta = lax.dot_general(xt, wp, (((1,), (1,)), ((), ())),
                         preferred_element_type=jnp.float32) + bp
    hre = hre_ref[...]
    him = him_ref[...]
    tre = tre_ref[...]
    tim = tim_ref[...]
    rre = rre_ref[...]
    rim = rim_ref[...]
    h = h_ref[...]
    t = t_ref[...]
    r = r_ref[...]
    c = -jnp.sum(rre * (hre * tre + him * tim) + rim * (hre * tim - him * tre),
                 axis=-1)
    s_tt = jnp.sum(h * t * r, axis=-1)
    s_it = jnp.sum(ha * t * r, axis=-1)
    s_ti = jnp.sum(h * ta * r, axis=-1)
    s_ii = jnp.sum(ha * ta * r, axis=-1)
    tm = tm_ref[0, ...]
    score = jnp.where(tm == 0, c - s_tt, 0.0)
    score = score + jnp.where(tm == 1, 2.0 * c - s_it - s_ti, 0.0)
    score = score + jnp.where(tm == 2, c - s_ii, 0.0)
    out_ref[0, ...] = score


@jax.jit
def _fused_score(xh, xt, hre, him, h, tre, tim, t, rre, rim, r, tm, wp, bp):
    row2 = lambda: pl.BlockSpec((BM, 2 * DIM), lambda i: (i, 0))
    row1 = lambda: pl.BlockSpec((BM, DIM), lambda i: (i, 0))
    grid_spec = pl.GridSpec(
        grid=(NB,),
        in_specs=[
            pl.BlockSpec((BM, VIS), lambda i: (i, 0)),
            pl.BlockSpec((BM, VIS), lambda i: (i, 0)),
            row1(), row1(), row2(),
            row1(), row1(), row2(),
            row1(), row1(), row2(),
            pl.BlockSpec((1, 1, BM), lambda i: (i, 0, 0)),
            pl.BlockSpec((2 * DIM, VIS), lambda i: (0, 0)),
            pl.BlockSpec((1, 2 * DIM), lambda i: (0, 0)),
        ],
        out_specs=pl.BlockSpec((1, 1, BM), lambda i: (i, 0, 0)),
    )
    out = pl.pallas_call(
        _score_block,
        grid_spec=grid_spec,
        out_shape=jax.ShapeDtypeStruct((NB, 1, BM), jnp.float32),
    )(xh, xt, hre, him, h, tre, tim, t, rre, rim, r,
      tm.reshape(NB, 1, BM), wp, bp.reshape(1, 2 * DIM))
    return out.reshape(B)


def kernel(batch_h, batch_t, batch_r, task_mode, mode,
           ent_re, ent_im, ent_emb, rel_re, rel_im, rel_emb,
           visual, Wp, bp):
    xh = jnp.take(visual, batch_h, axis=0)
    xt = jnp.take(visual, batch_t, axis=0)
    hre = jnp.take(ent_re, batch_h, axis=0)
    him = jnp.take(ent_im, batch_h, axis=0)
    h = jnp.take(ent_emb, batch_h, axis=0)
    tre = jnp.take(ent_re, batch_t, axis=0)
    tim = jnp.take(ent_im, batch_t, axis=0)
    t = jnp.take(ent_emb, batch_t, axis=0)
    rre = jnp.take(rel_re, batch_r, axis=0)
    rim = jnp.take(rel_im, batch_r, axis=0)
    r = jnp.take(rel_emb, batch_r, axis=0)
    return _fused_score(xh, xt, hre, him, h, tre, tim, t, rre, rim, r,
                        task_mode, Wp, bp)


# speedup vs baseline: 2.0656x; 1.4648x over previous
import functools

import jax
import jax.numpy as jnp
from jax import lax
from jax.experimental import pallas as pl
from jax.experimental.pallas import tpu as pltpu

B = 16384
DIM = 128
VIS = 4096
BM = 256
NB = B // BM


def _score_block(bh_ref, bt_ref, tms_ref,
                 visual_ref,
                 hre_ref, him_ref, h_ref, tre_ref, tim_ref, t_ref,
                 rre_ref, rim_ref, r_ref, tm_ref, wp_ref, bp_ref,
                 out_ref,
                 xh_buf, xt_buf, sem):
    i = pl.program_id(0)
    nb = pl.num_programs(0)

    def issue(block, slot):
        def body(j, carry):
            row = block * BM + j

            @pl.when(tms_ref[row] != 0)
            def _():
                pltpu.make_async_copy(
                    visual_ref.at[bh_ref[row]], xh_buf.at[slot, j],
                    sem.at[slot, 0]).start()
                pltpu.make_async_copy(
                    visual_ref.at[bt_ref[row]], xt_buf.at[slot, j],
                    sem.at[slot, 1]).start()
            return carry
        lax.fori_loop(0, BM, body, 0, unroll=8)

    def wait(block, slot):
        def body(j, carry):
            row = block * BM + j

            @pl.when(tms_ref[row] != 0)
            def _():
                pltpu.make_async_copy(
                    visual_ref.at[bh_ref[row]], xh_buf.at[slot, j],
                    sem.at[slot, 0]).wait()
                pltpu.make_async_copy(
                    visual_ref.at[bt_ref[row]], xt_buf.at[slot, j],
                    sem.at[slot, 1]).wait()
            return carry
        lax.fori_loop(0, BM, body, 0, unroll=8)

    @pl.when(i == 0)
    def _():
        issue(0, 0)

    @pl.when(i + 1 < nb)
    def _():
        issue(i + 1, (i + 1) % 2)

    slot = i % 2
    wait(i, slot)

    xh = xh_buf[slot]
    xt = xt_buf[slot]
    wp = wp_ref[...]
    bp = bp_ref[...]
    ha = lax.dot_general(xh, wp, (((1,), (1,)), ((), ())),
                         preferred_element_type=jnp.float32) + bp
    ta = lax.dot_general(xt, wp, (((1,), (1,)), ((), ())),
                         preferred_element_type=jnp.float32) + bp
    hre = hre_ref[...]
    him = him_ref[...]
    tre = tre_ref[...]
    tim = tim_ref[...]
    rre = rre_ref[...]
    rim = rim_ref[...]
    h = h_ref[...]
    t = t_ref[...]
    r = r_ref[...]
    c = -jnp.sum(rre * (hre * tre + him * tim) + rim * (hre * tim - him * tre),
                 axis=-1)
    s_tt = jnp.sum(h * t * r, axis=-1)
    s_it = jnp.sum(ha * t * r, axis=-1)
    s_ti = jnp.sum(h * ta * r, axis=-1)
    s_ii = jnp.sum(ha * ta * r, axis=-1)
    tm = tm_ref[0, ...]
    score = jnp.where(tm == 0, c - s_tt, 0.0)
    score = score + jnp.where(tm == 1, 2.0 * c - s_it - s_ti, 0.0)
    score = score + jnp.where(tm == 2, c - s_ii, 0.0)
    out_ref[0, ...] = score


@jax.jit
def _fused_score(bh, bt, visual, hre, him, h, tre, tim, t, rre, rim, r,
                 tm, wp, bp):
    row2 = lambda: pl.BlockSpec((BM, 2 * DIM), lambda i, *_: (i, 0))
    row1 = lambda: pl.BlockSpec((BM, DIM), lambda i, *_: (i, 0))
    grid_spec = pltpu.PrefetchScalarGridSpec(
        num_scalar_prefetch=3,
        grid=(NB,),
        in_specs=[
            pl.BlockSpec(memory_space=pltpu.MemorySpace.HBM),
            row1(), row1(), row2(),
            row1(), row1(), row2(),
            row1(), row1(), row2(),
            pl.BlockSpec((1, 1, BM), lambda i, *_: (i, 0, 0)),
            pl.BlockSpec((2 * DIM, VIS), lambda i, *_: (0, 0)),
            pl.BlockSpec((1, 2 * DIM), lambda i, *_: (0, 0)),
        ],
        out_specs=pl.BlockSpec((1, 1, BM), lambda i, *_: (i, 0, 0)),
        scratch_shapes=[
            pltpu.VMEM((2, BM, VIS), jnp.float32),
            pltpu.VMEM((2, BM, VIS), jnp.float32),
            pltpu.SemaphoreType.DMA((2, 2)),
        ],
    )
    out = pl.pallas_call(
        _score_block,
        grid_spec=grid_spec,
        out_shape=jax.ShapeDtypeStruct((NB, 1, BM), jnp.float32),
    )(bh, bt, tm, visual, hre, him, h, tre, tim, t, rre, rim, r,
      tm.reshape(NB, 1, BM), wp, bp.reshape(1, 2 * DIM))
    return out.reshape(B)


def kernel(batch_h, batch_t, batch_r, task_mode, mode,
           ent_re, ent_im, ent_emb, rel_re, rel_im, rel_emb,
           visual, Wp, bp):
    hre = jnp.take(ent_re, batch_h, axis=0)
    him = jnp.take(ent_im, batch_h, axis=0)
    h = jnp.take(ent_emb, batch_h, axis=0)
    tre = jnp.take(ent_re, batch_t, axis=0)
    tim = jnp.take(ent_im, batch_t, axis=0)
    t = jnp.take(ent_emb, batch_t, axis=0)
    rre = jnp.take(rel_re, batch_r, axis=0)
    rim = jnp.take(rel_im, batch_r, axis=0)
    r = jnp.take(rel_emb, batch_r, axis=0)
    return _fused_score(batch_h, batch_t, visual, hre, him, h, tre, tim, t,
                        rre, rim, r, task_mode, Wp, bp)
